# baseline (speedup 1.0000x reference)
import functools

import jax
import jax.numpy as jnp
from jax import lax
from jax.experimental import pallas as pl
from jax.experimental.pallas import tpu as pltpu
from jax.experimental.pallas import tpu_sc as plsc

_EPS = 1e-12
_NC, _NS = 2, 16
_NW = _NC * _NS
_IDX_CHUNK = 64
_TN = 2048


def _sc_gather(word_emb, ids, base, n_tok):
    d = word_emb.shape[1]
    per_w = n_tok // _NW
    n_sub = (per_w + _IDX_CHUNK - 1) // _IDX_CHUNK
    mesh = plsc.VectorSubcoreMesh(core_axis_name="c", subcore_axis_name="s")

    @functools.partial(
        pl.kernel,
        mesh=mesh,
        out_type=jax.ShapeDtypeStruct((n_tok, d), word_emb.dtype),
        scratch_types=[
            pltpu.VMEM((per_w,), jnp.int32),
            pltpu.VMEM((per_w, d), word_emb.dtype),
            pltpu.SemaphoreType.DMA,
            pltpu.SemaphoreType.DMA,
        ],
    )
    def gather_kernel(table_hbm, idx_hbm, out_hbm, idx_v, rows_v, gsem, wsem):
        wid = lax.axis_index("s") * _NC + lax.axis_index("c")
        w_base = base + wid * per_w
        pltpu.sync_copy(idx_hbm.at[pl.ds(w_base, per_w)], idx_v)
        copies = []
        for j in range(n_sub):
            lo = j * _IDX_CHUNK
            sz = min(_IDX_CHUNK, per_w - lo)
            copies.append(
                pltpu.async_copy(
                    table_hbm.at[idx_v.at[pl.ds(lo, sz)]],
                    rows_v.at[pl.ds(lo, sz)],
                    gsem,
                )
            )
        writes = []
        for j, c in enumerate(copies):
            c.wait()
            lo = j * _IDX_CHUNK
            sz = min(_IDX_CHUNK, per_w - lo)
            writes.append(
                pltpu.async_copy(
                    rows_v.at[pl.ds(lo, sz)],
                    out_hbm.at[pl.ds(wid * per_w + lo, sz)],
                    wsem,
                )
            )
        for w in writes:
            w.wait()

    return gather_kernel(word_emb, ids)


def _dense_body(g_ref, pos_ref, type_ref, gam_ref, bet_ref, w_ref, b_ref, o_ref):
    x = g_ref[...] + pos_ref[...] + type_ref[0:1, :]
    mu = jnp.mean(x, axis=1, keepdims=True)
    xc = x - mu
    var = jnp.mean(xc * xc, axis=1, keepdims=True)
    nrm = xc * lax.rsqrt(var + _EPS) * gam_ref[...] + bet_ref[...]
    o_ref[...] = (
        jnp.dot(nrm, w_ref[...], preferred_element_type=jnp.float32) + b_ref[...]
    )


def _dense_body_acc(g_ref, pos_ref, type_ref, gam_ref, bet_ref, w_ref, b_ref,
                    buf_ref, o_ref):
    del buf_ref
    _dense_body(g_ref, pos_ref, type_ref, gam_ref, bet_ref, w_ref, b_ref, o_ref)


def _tc_dense(gathered, pos_emb, type_emb, ln_gamma, ln_beta, w_map, b_map,
              seq_len, total_n, block_off, buf):
    n, d = gathered.shape
    h = w_map.shape[1]
    blocks_per_seq = seq_len // _TN
    n_batch = n // seq_len
    grid = (blocks_per_seq, n_batch)
    in_specs = [
        pl.BlockSpec((_TN, d), lambda p, b: (b * blocks_per_seq + p, 0)),
        pl.BlockSpec((_TN, d), lambda p, b: (p, 0)),
        pl.BlockSpec(type_emb.shape, lambda p, b: (0, 0)),
        pl.BlockSpec((1, d), lambda p, b: (0, 0)),
        pl.BlockSpec((1, d), lambda p, b: (0, 0)),
        pl.BlockSpec((d, h), lambda p, b: (0, 0)),
        pl.BlockSpec((1, h), lambda p, b: (0, 0)),
    ]
    args = [
        gathered,
        pos_emb,
        type_emb,
        ln_gamma.reshape(1, d),
        ln_beta.reshape(1, d),
        w_map,
        b_map.reshape(1, h),
    ]
    body = _dense_body
    io_aliases = {}
    if buf is not None:
        in_specs.append(pl.BlockSpec(memory_space=pl.ANY))
        args.append(buf)
        body = _dense_body_acc
        io_aliases = {7: 0}
    return pl.pallas_call(
        body,
        grid=grid,
        in_specs=in_specs,
        out_specs=pl.BlockSpec(
            (_TN, h), lambda p, b: (block_off + b * blocks_per_seq + p, 0)
        ),
        out_shape=jax.ShapeDtypeStruct((total_n, h), jnp.float32),
        input_output_aliases=io_aliases,
        compiler_params=pltpu.CompilerParams(
            dimension_semantics=("parallel", "parallel"),
        ),
    )(*args)


def kernel(input_ids, word_emb, type_emb, pos_emb, ln_gamma, ln_beta, W_map, b_map):
    b, s = input_ids.shape
    n = b * s
    h = W_map.shape[1]
    ids = input_ids.reshape(n).astype(jnp.int32)
    g = _sc_gather(word_emb, ids, 0, n)
    out = _tc_dense(
        g, pos_emb, type_emb, ln_gamma, ln_beta, W_map, b_map, s, n, 0, None
    )
    return out.reshape(b, s, h)


# speedup vs baseline: 1.0048x; 1.0048x over previous
import functools

import jax
import jax.numpy as jnp
from jax import lax
from jax.experimental import pallas as pl
from jax.experimental.pallas import tpu as pltpu
from jax.experimental.pallas import tpu_sc as plsc

_EPS = 1e-12
_NC, _NS = 2, 16
_NW = _NC * _NS
_IDX_CHUNK = 128
_TN = 2048


def _sc_gather(word_emb, ids, base, n_tok):
    d = word_emb.shape[1]
    per_w = n_tok // _NW
    n_sub = (per_w + _IDX_CHUNK - 1) // _IDX_CHUNK
    mesh = plsc.VectorSubcoreMesh(core_axis_name="c", subcore_axis_name="s")

    @functools.partial(
        pl.kernel,
        mesh=mesh,
        out_type=jax.ShapeDtypeStruct((n_tok, d), word_emb.dtype),
        scratch_types=[
            pltpu.VMEM((per_w,), jnp.int32),
            pltpu.VMEM((per_w, d), word_emb.dtype),
            pltpu.SemaphoreType.DMA,
            pltpu.SemaphoreType.DMA,
        ],
    )
    def gather_kernel(table_hbm, idx_hbm, out_hbm, idx_v, rows_v, gsem, wsem):
        wid = lax.axis_index("s") * _NC + lax.axis_index("c")
        w_base = base + wid * per_w
        pltpu.sync_copy(idx_hbm.at[pl.ds(w_base, per_w)], idx_v)
        copies = []
        for j in range(n_sub):
            lo = j * _IDX_CHUNK
            sz = min(_IDX_CHUNK, per_w - lo)
            copies.append(
                pltpu.async_copy(
                    table_hbm.at[idx_v.at[pl.ds(lo, sz)]],
                    rows_v.at[pl.ds(lo, sz)],
                    gsem,
                )
            )
        writes = []
        for j, c in enumerate(copies):
            c.wait()
            lo = j * _IDX_CHUNK
            sz = min(_IDX_CHUNK, per_w - lo)
            writes.append(
                pltpu.async_copy(
                    rows_v.at[pl.ds(lo, sz)],
                    out_hbm.at[pl.ds(wid * per_w + lo, sz)],
                    wsem,
                )
            )
        for w in writes:
            w.wait()

    return gather_kernel(word_emb, ids)


def _dense_body(g_ref, pos_ref, type_ref, gam_ref, bet_ref, w_ref, b_ref, o_ref):
    x = g_ref[...] + pos_ref[...] + type_ref[0:1, :]
    mu = jnp.mean(x, axis=1, keepdims=True)
    xc = x - mu
    var = jnp.mean(xc * xc, axis=1, keepdims=True)
    nrm = xc * lax.rsqrt(var + _EPS) * gam_ref[...] + bet_ref[...]
    o_ref[...] = (
        jnp.dot(nrm, w_ref[...], preferred_element_type=jnp.float32) + b_ref[...]
    )


def _dense_body_acc(g_ref, pos_ref, type_ref, gam_ref, bet_ref, w_ref, b_ref,
                    buf_ref, o_ref):
    del buf_ref
    _dense_body(g_ref, pos_ref, type_ref, gam_ref, bet_ref, w_ref, b_ref, o_ref)


def _tc_dense(gathered, pos_emb, type_emb, ln_gamma, ln_beta, w_map, b_map,
              seq_len, total_n, block_off, buf):
    n, d = gathered.shape
    h = w_map.shape[1]
    blocks_per_seq = seq_len // _TN
    n_batch = n // seq_len
    grid = (blocks_per_seq, n_batch)
    in_specs = [
        pl.BlockSpec((_TN, d), lambda p, b: (b * blocks_per_seq + p, 0)),
        pl.BlockSpec((_TN, d), lambda p, b: (p, 0)),
        pl.BlockSpec(type_emb.shape, lambda p, b: (0, 0)),
        pl.BlockSpec((1, d), lambda p, b: (0, 0)),
        pl.BlockSpec((1, d), lambda p, b: (0, 0)),
        pl.BlockSpec((d, h), lambda p, b: (0, 0)),
        pl.BlockSpec((1, h), lambda p, b: (0, 0)),
    ]
    args = [
        gathered,
        pos_emb,
        type_emb,
        ln_gamma.reshape(1, d),
        ln_beta.reshape(1, d),
        w_map,
        b_map.reshape(1, h),
    ]
    body = _dense_body
    io_aliases = {}
    if buf is not None:
        in_specs.append(pl.BlockSpec(memory_space=pl.ANY))
        args.append(buf)
        body = _dense_body_acc
        io_aliases = {7: 0}
    return pl.pallas_call(
        body,
        grid=grid,
        in_specs=in_specs,
        out_specs=pl.BlockSpec(
            (_TN, h), lambda p, b: (block_off + b * blocks_per_seq + p, 0)
        ),
        out_shape=jax.ShapeDtypeStruct((total_n, h), jnp.float32),
        input_output_aliases=io_aliases,
        compiler_params=pltpu.CompilerParams(
            dimension_semantics=("parallel", "parallel"),
        ),
    )(*args)


def kernel(input_ids, word_emb, type_emb, pos_emb, ln_gamma, ln_beta, W_map, b_map):
    b, s = input_ids.shape
    n = b * s
    h = W_map.shape[1]
    ids = input_ids.reshape(n).astype(jnp.int32)
    g = _sc_gather(word_emb, ids, 0, n)
    out = _tc_dense(
        g, pos_emb, type_emb, ln_gamma, ln_beta, W_map, b_map, s, n, 0, None
    )
    return out.reshape(b, s, h)


# speedup vs baseline: 1.0055x; 1.0007x over previous
import functools

import jax
import jax.numpy as jnp
from jax import lax
from jax.experimental import pallas as pl
from jax.experimental.pallas import tpu as pltpu
from jax.experimental.pallas import tpu_sc as plsc

_EPS = 1e-12
_NC, _NS = 2, 16
_NW = _NC * _NS
_IDX_CHUNK = 128
_TN = 2048


def _sc_gather(word_emb, ids2d):
    d = word_emb.shape[1]
    n_tok = ids2d.shape[0] * ids2d.shape[1]
    s = ids2d.shape[1]
    per_w = n_tok // _NW
    n_sub = (per_w + _IDX_CHUNK - 1) // _IDX_CHUNK
    mesh = plsc.VectorSubcoreMesh(core_axis_name="c", subcore_axis_name="s")

    @functools.partial(
        pl.kernel,
        mesh=mesh,
        out_type=jax.ShapeDtypeStruct((n_tok, d), word_emb.dtype),
        scratch_types=[
            pltpu.VMEM((per_w,), jnp.int32),
            pltpu.VMEM((per_w, d), word_emb.dtype),
            pltpu.SemaphoreType.DMA,
            pltpu.SemaphoreType.DMA,
        ],
    )
    def gather_kernel(table_hbm, idx_hbm, out_hbm, idx_v, rows_v, gsem, wsem):
        wid = lax.axis_index("s") * _NC + lax.axis_index("c")
        w_base = wid * per_w
        pltpu.sync_copy(idx_hbm.at[w_base // s, pl.ds(w_base % s, per_w)], idx_v)
        copies = []
        for j in range(n_sub):
            lo = j * _IDX_CHUNK
            sz = min(_IDX_CHUNK, per_w - lo)
            copies.append(
                pltpu.async_copy(
                    table_hbm.at[idx_v.at[pl.ds(lo, sz)]],
                    rows_v.at[pl.ds(lo, sz)],
                    gsem,
                )
            )
        writes = []
        for j, c in enumerate(copies):
            c.wait()
            lo = j * _IDX_CHUNK
            sz = min(_IDX_CHUNK, per_w - lo)
            writes.append(
                pltpu.async_copy(
                    rows_v.at[pl.ds(lo, sz)],
                    out_hbm.at[pl.ds(wid * per_w + lo, sz)],
                    wsem,
                )
            )
        for w in writes:
            w.wait()

    return gather_kernel(word_emb, ids2d)


def _dense_body(g_ref, pos_ref, type_ref, gam_ref, bet_ref, w_ref, b_ref, o_ref):
    x = g_ref[...] + pos_ref[...] + type_ref[0:1, :]
    mu = jnp.mean(x, axis=1, keepdims=True)
    xc = x - mu
    var = jnp.mean(xc * xc, axis=1, keepdims=True)
    nrm = xc * lax.rsqrt(var + _EPS) * gam_ref[...] + bet_ref[...]
    o_ref[...] = (
        jnp.dot(nrm, w_ref[...], preferred_element_type=jnp.float32) + b_ref[...]
    )


def _dense_body_acc(g_ref, pos_ref, type_ref, gam_ref, bet_ref, w_ref, b_ref,
                    buf_ref, o_ref):
    del buf_ref
    _dense_body(g_ref, pos_ref, type_ref, gam_ref, bet_ref, w_ref, b_ref, o_ref)


def _tc_dense(gathered, pos_emb, type_emb, ln_gamma, ln_beta, w_map, b_map,
              seq_len, total_n, block_off, buf):
    n, d = gathered.shape
    h = w_map.shape[1]
    blocks_per_seq = seq_len // _TN
    n_batch = n // seq_len
    grid = (blocks_per_seq, n_batch)
    in_specs = [
        pl.BlockSpec((_TN, d), lambda p, b: (b * blocks_per_seq + p, 0)),
        pl.BlockSpec((_TN, d), lambda p, b: (p, 0)),
        pl.BlockSpec(type_emb.shape, lambda p, b: (0, 0)),
        pl.BlockSpec((1, d), lambda p, b: (0, 0)),
        pl.BlockSpec((1, d), lambda p, b: (0, 0)),
        pl.BlockSpec((d, h), lambda p, b: (0, 0)),
        pl.BlockSpec((1, h), lambda p, b: (0, 0)),
    ]
    args = [
        gathered,
        pos_emb,
        type_emb,
        ln_gamma.reshape(1, d),
        ln_beta.reshape(1, d),
        w_map,
        b_map.reshape(1, h),
    ]
    body = _dense_body
    io_aliases = {}
    if buf is not None:
        in_specs.append(pl.BlockSpec(memory_space=pl.ANY))
        args.append(buf)
        body = _dense_body_acc
        io_aliases = {7: 0}
    return pl.pallas_call(
        body,
        grid=grid,
        in_specs=in_specs,
        out_specs=pl.BlockSpec(
            (_TN, h), lambda p, b: (block_off + b * blocks_per_seq + p, 0)
        ),
        out_shape=jax.ShapeDtypeStruct((total_n, h), jnp.float32),
        input_output_aliases=io_aliases,
        compiler_params=pltpu.CompilerParams(
            dimension_semantics=("parallel", "parallel"),
        ),
    )(*args)


def kernel(input_ids, word_emb, type_emb, pos_emb, ln_gamma, ln_beta, W_map, b_map):
    b, s = input_ids.shape
    n = b * s
    h = W_map.shape[1]
    ids = input_ids.astype(jnp.int32)
    if s % (n // _NW):
        ids = ids.reshape(1, n)
    g = _sc_gather(word_emb, ids)
    out = _tc_dense(
        g, pos_emb, type_emb, ln_gamma, ln_beta, W_map, b_map, s, n, 0, None
    )
    return out.reshape(b, s, h)
